# baseline (speedup 1.0000x reference)
import functools

import jax
import jax.numpy as jnp
from jax import lax
from jax.experimental import pallas as pl
from jax.experimental.pallas import tpu as pltpu
from jax.experimental.pallas import tpu_sc as plsc

_D = 64
_C = 20
_VC = 2048
_IDX_CHUNK = 128


def _pool_sc(item_id, ctx_ids_flat, item_table, ctx_table):
    B = item_id.shape[0]
    info = plsc.get_sparse_core_info()
    nw = info.num_cores * info.num_subcores
    assert B % nw == 0
    bpw = B // nw
    nci = bpw * _C
    assert nci % _IDX_CHUNK == 0
    n_gather = nci // _IDX_CHUNK

    item_idx2 = item_id.reshape(nw, bpw)
    ctx_idx3 = ctx_ids_flat.reshape(nw, nci // _IDX_CHUNK, _IDX_CHUNK)

    mesh = plsc.VectorSubcoreMesh(
        core_axis_name="c", subcore_axis_name="s",
        num_cores=info.num_cores)

    @functools.partial(
        pl.kernel,
        out_type=jax.ShapeDtypeStruct((B, _D), jnp.float32),
        mesh=mesh,
        scratch_types=[
            pltpu.VMEM((bpw,), jnp.int32),
            pltpu.VMEM((n_gather, _IDX_CHUNK), jnp.int32),
            pltpu.VMEM((bpw, _D), jnp.float32),
            pltpu.VMEM((nci, _D), jnp.float32),
            pltpu.VMEM((bpw, _D), jnp.float32),
            pltpu.SemaphoreType.DMA,
        ],
        compiler_params=pltpu.CompilerParams(use_tc_tiling_on_sc=False),
    )
    def pool(item_idx_hbm, ctx_idx_hbm, item_tab_hbm, ctx_tab_hbm, out_hbm,
             idxi_v, idxc_v, item_rows, ctx_rows, out_v, sem):
        wid = lax.axis_index("s") * info.num_cores + lax.axis_index("c")
        base = wid * bpw
        pltpu.sync_copy(item_idx_hbm.at[wid], idxi_v)
        pltpu.sync_copy(ctx_idx_hbm.at[wid], idxc_v)
        pltpu.async_copy(item_tab_hbm.at[idxi_v], item_rows, sem).wait()
        for g in range(n_gather):
            pltpu.async_copy(
                ctx_tab_hbm.at[idxc_v.at[g]],
                ctx_rows.at[pl.ds(g * _IDX_CHUNK, _IDX_CHUNK)],
                sem,
            ).wait()
        scale = 1.0 / (_C + 1)

        def body(b, carry):
            for k in range(_D // 16):
                acc = item_rows[b, pl.ds(k * 16, 16)]
                for c in range(_C):
                    acc = acc + ctx_rows[b * _C + c, pl.ds(k * 16, 16)]
                out_v[b, pl.ds(k * 16, 16)] = acc * scale
            return carry

        lax.fori_loop(0, bpw, body, 0)
        pltpu.sync_copy(out_v, out_hbm.at[pl.ds(base, bpw)])

    return pool(item_idx2, ctx_idx3, item_table, ctx_table)


def _sumexp_body(v_total, x_ref, w_ref, b_ref, s_ref):
    j = pl.program_id(0)
    logits = lax.dot_general(
        x_ref[...], w_ref[...], (((1,), (1,)), ((), ())),
        preferred_element_type=jnp.float32,
    ) + b_ref[...]
    col = j * _VC + lax.broadcasted_iota(jnp.int32, logits.shape, 1)
    e = jnp.where(col < v_total, jnp.exp(logits), 0.0)

    @pl.when(j == 0)
    def _():
        s_ref[...] = jnp.zeros_like(s_ref)

    s_ref[...] += jnp.sum(e, axis=1, keepdims=True)


def _out_body(x_ref, w_ref, b_ref, s_ref, o_ref):
    logits = lax.dot_general(
        x_ref[...], w_ref[...], (((1,), (1,)), ((), ())),
        preferred_element_type=jnp.float32,
    ) + b_ref[...]
    o_ref[...] = logits - jnp.log(s_ref[...])


def _logsoftmax_tc(pooled, W, b):
    B = pooled.shape[0]
    V = W.shape[0]
    n_chunks = pl.cdiv(V, _VC)
    b2 = b.reshape(1, V)

    x_spec = pl.BlockSpec((B, _D), lambda j: (0, 0))
    w_spec = pl.BlockSpec((_VC, _D), lambda j: (j, 0))
    b_spec = pl.BlockSpec((1, _VC), lambda j: (0, j))
    s_spec = pl.BlockSpec((B, 1), lambda j: (0, 0))

    s = pl.pallas_call(
        functools.partial(_sumexp_body, V),
        grid=(n_chunks,),
        in_specs=[x_spec, w_spec, b_spec],
        out_specs=s_spec,
        out_shape=jax.ShapeDtypeStruct((B, 1), jnp.float32),
        compiler_params=pltpu.CompilerParams(
            dimension_semantics=("arbitrary",)),
    )(pooled, W, b2)

    out = pl.pallas_call(
        _out_body,
        grid=(n_chunks,),
        in_specs=[x_spec, w_spec, b_spec, s_spec],
        out_specs=pl.BlockSpec((B, _VC), lambda j: (0, j)),
        out_shape=jax.ShapeDtypeStruct((B, V), jnp.float32),
        compiler_params=pltpu.CompilerParams(
            dimension_semantics=("arbitrary",)),
    )(pooled, W, b2, s)
    return out


def kernel(context_ids, item_id, negative_samples_ids, item_table, ctx_table,
           W, b):
    del negative_samples_ids
    ctx_flat = context_ids.astype(jnp.int32).T.reshape(-1)
    pooled = _pool_sc(item_id.astype(jnp.int32), ctx_flat, item_table,
                      ctx_table)
    return _logsoftmax_tc(pooled, W, b)
